# baseline (speedup 1.0000x reference)
import jax
import jax.numpy as jnp
from jax import lax
from jax.experimental import pallas as pl
from jax.experimental.pallas import tpu as pltpu
from jax.experimental.pallas import tpu_sc as plsc

N = 10000
D = 128
E = 320000

NP = 10016
RB = 1000
NBLK = N // RB
DE = 144
DE_AL = 129
NT = 32
BATCH = 96
NBATCH = 108
EPT = NBATCH * BATCH
EPAD = NT * EPT
RPT = NP // 16


def _emit_g(g_ref, alad_ref, g, as_ref, ad_ref):
    g_ref[:, pl.ds(0, 128)] = g
    al = jnp.sum(g * as_ref[...], axis=1)
    lane = lax.broadcasted_iota(jnp.int32, (RB, 16), 1)
    g_ref[:, pl.ds(128, 16)] = (jnp.where(lane == 0, 1.0, 0.0)
                                + jnp.where(lane == 1, al[:, None], 0.0))
    alad_ref[0, 0, :] = al
    alad_ref[0, 1, :] = jnp.sum(g * ad_ref[...], axis=1)


def _enc_body(x_ref, encW_ref, encb_ref, W0_ref, as_ref, ad_ref,
              h_ref, g_ref, alad_ref):
    h = jnp.dot(x_ref[...], encW_ref[...],
                preferred_element_type=jnp.float32) + encb_ref[...]
    h_ref[...] = h
    g = jnp.dot(h, W0_ref[...], preferred_element_type=jnp.float32)
    _emit_g(g_ref, alad_ref, g, as_ref, ad_ref)


def _post_layer(h2p_ref, hin_ref, bi_ref, lnw_ref, lnb_ref):
    num = h2p_ref[0, :, pl.ds(0, 128)] + h2p_ref[1, :, pl.ds(0, 128)]
    den = h2p_ref[0, :, pl.ds(128, 1)] + h2p_ref[1, :, pl.ds(128, 1)]
    h2 = num / (den + 1e-16) + bi_ref[...]
    mu = jnp.mean(h2, axis=1, keepdims=True)
    zc = h2 - mu
    var = jnp.mean(zc * zc, axis=1, keepdims=True)
    h2n = zc / jnp.sqrt(var + 1e-5) * lnw_ref[...] + lnb_ref[...]
    return jnp.maximum(h2n, 0.0) + hin_ref[...]


def _mid_body(h2p_ref, hin_ref, bi_ref, lnw_ref, lnb_ref, Wn_ref, as_ref,
              ad_ref, hout_ref, g_ref, alad_ref):
    hout = _post_layer(h2p_ref, hin_ref, bi_ref, lnw_ref, lnb_ref)
    hout_ref[...] = hout
    g = jnp.dot(hout, Wn_ref[...], preferred_element_type=jnp.float32)
    _emit_g(g_ref, alad_ref, g, as_ref, ad_ref)


def _fin_body(h2p_ref, hin_ref, bi_ref, lnw_ref, lnb_ref, decW_ref, decb_ref,
              out_ref):
    hout = _post_layer(h2p_ref, hin_ref, bi_ref, lnw_ref, lnb_ref)
    logits = jnp.dot(hout, decW_ref[...],
                     preferred_element_type=jnp.float32) + decb_ref[...]
    sg = jax.nn.sigmoid(logits)

    @pl.when(pl.program_id(0) == 0)
    def _():
        out_ref[...] = jnp.zeros_like(out_ref)

    out_ref[...] += jnp.sum(sg, axis=0, keepdims=True)


_full = lambda shape: pl.BlockSpec(shape, lambda i: tuple(0 for _ in shape))

_enc_call = pl.pallas_call(
    _enc_body,
    grid=(NBLK,),
    in_specs=[
        pl.BlockSpec((RB, D), lambda i: (i, 0)),
        _full((D, D)), _full((1, D)), _full((D, D)), _full((1, D)),
        _full((1, D)),
    ],
    out_specs=[
        pl.BlockSpec((RB, D), lambda i: (i, 0)),
        pl.BlockSpec((RB, DE), lambda i: (i, 0)),
        pl.BlockSpec((1, 2, RB), lambda i: (i, 0, 0)),
    ],
    out_shape=[
        jax.ShapeDtypeStruct((N, D), jnp.float32),
        jax.ShapeDtypeStruct((NP, DE), jnp.float32),
        jax.ShapeDtypeStruct((NBLK, 2, RB), jnp.float32),
    ],
)

_mid_call = pl.pallas_call(
    _mid_body,
    grid=(NBLK,),
    in_specs=[
        pl.BlockSpec((2, RB, DE), lambda i: (0, i, 0)),
        pl.BlockSpec((RB, D), lambda i: (i, 0)),
        _full((1, D)), _full((1, D)), _full((1, D)), _full((D, D)),
        _full((1, D)), _full((1, D)),
    ],
    out_specs=[
        pl.BlockSpec((RB, D), lambda i: (i, 0)),
        pl.BlockSpec((RB, DE), lambda i: (i, 0)),
        pl.BlockSpec((1, 2, RB), lambda i: (i, 0, 0)),
    ],
    out_shape=[
        jax.ShapeDtypeStruct((N, D), jnp.float32),
        jax.ShapeDtypeStruct((NP, DE), jnp.float32),
        jax.ShapeDtypeStruct((NBLK, 2, RB), jnp.float32),
    ],
)

_fin_call = pl.pallas_call(
    _fin_body,
    grid=(NBLK,),
    in_specs=[
        pl.BlockSpec((2, RB, DE), lambda i: (0, i, 0)),
        pl.BlockSpec((RB, D), lambda i: (i, 0)),
        _full((1, D)), _full((1, D)), _full((1, D)), _full((D, D)),
        _full((1, D)),
    ],
    out_specs=pl.BlockSpec((1, D), lambda i: (0, 0)),
    out_shape=jax.ShapeDtypeStruct((1, D), jnp.float32),
)


def _sc_body(g_hbm, alad_hbm, src_hbm, dst_hbm, h2p_hbm,
             ad_v, src_c, dst_c, rows_v, h2_sh, gsem, ssem, isem):
    c = lax.axis_index("c")
    s = lax.axis_index("s")
    wid = s * 2 + c
    row0 = s * RPT

    zv = jnp.zeros((16,), jnp.float32)

    def _z(i, carry):
        for rb in range(2):
            for k in range(DE // 16):
                rows_v[rb, i, pl.ds(k * 16, 16)] = zv
        return carry

    lax.fori_loop(0, BATCH, _z, 0)
    for k in range(RPT // BATCH):
        pltpu.sync_copy(rows_v.at[0], h2_sh.at[pl.ds(row0 + k * BATCH, BATCH)])
    rem = RPT % BATCH
    pltpu.sync_copy(rows_v.at[0, pl.ds(0, rem)],
                    h2_sh.at[pl.ds(row0 + RPT - rem, rem)])

    for k in range(NBLK):
        pltpu.sync_copy(alad_hbm.at[k, 1], ad_v.at[pl.ds(k * RB, RB)])
    ad_v[pl.ds(N, NP - N)] = jnp.full((16,), -1e30, jnp.float32)
    plsc.subcore_barrier()

    pltpu.sync_copy(src_hbm.at[wid, 0], src_c.at[0])
    pltpu.sync_copy(dst_hbm.at[wid, 0], dst_c.at[0])
    pltpu.async_copy(g_hbm.at[src_c.at[0]], rows_v.at[0], gsem)
    pltpu.async_copy(src_hbm.at[wid, 1], src_c.at[1], isem)
    pltpu.async_copy(dst_hbm.at[wid, 1], dst_c.at[1], isem)

    col1 = jnp.full((16,), DE_AL, jnp.int32)
    lanes = lax.iota(jnp.int32, 16)

    def _batch(b, carry):
        rb = lax.rem(b, 2)
        nrb = lax.rem(b + 1, 2)
        i3 = lax.rem(b, 3)
        n3 = lax.rem(b + 1, 3)

        @pl.when(b + 1 < NBATCH)
        def _():
            pltpu.make_async_copy(src_hbm.at[wid, 0], src_c.at[n3], isem).wait()
            pltpu.make_async_copy(dst_hbm.at[wid, 0], dst_c.at[n3], isem).wait()

        @pl.when(b >= 1)
        def _():
            pltpu.make_async_copy(rows_v.at[nrb],
                                  h2_sh.at[dst_c.at[0]], ssem).wait()

        @pl.when(b + 1 < NBATCH)
        def _():
            pltpu.async_copy(g_hbm.at[src_c.at[n3]], rows_v.at[nrb], gsem)

        @pl.when(b + 2 < NBATCH)
        def _():
            f3 = lax.rem(b + 2, 3)
            pltpu.async_copy(src_hbm.at[wid, b + 2], src_c.at[f3], isem)
            pltpu.async_copy(dst_hbm.at[wid, b + 2], dst_c.at[f3], isem)

        pltpu.make_async_copy(g_hbm.at[src_c.at[i3]], rows_v.at[rb],
                              gsem).wait()

        rbv = jnp.full((16,), 0, jnp.int32) + rb

        def _scale(g16, inner):
            rids = g16 * 16 + lanes
            alg = plsc.load_gather(rows_v, [rbv, rids, col1])
            dv = dst_c[i3, pl.ds(g16 * 16, 16)]
            t = alg + plsc.load_gather(ad_v, [dv])
            ex = jnp.exp(jnp.maximum(t, 0.2 * t))
            for kk in range(16):
                sc = ex[kk]
                row = g16 * 16 + kk
                for k in range(DE // 16):
                    sl = pl.ds(k * 16, 16)
                    rows_v[rb, row, sl] = rows_v[rb, row, sl] * sc
            return inner

        lax.fori_loop(0, BATCH // 16, _scale, 0)
        pltpu.async_copy(rows_v.at[rb], h2_sh.at[dst_c.at[i3]], ssem,
                         add=True)
        return carry

    lax.fori_loop(0, NBATCH, _batch, 0)
    pltpu.make_async_copy(rows_v.at[0], h2_sh.at[dst_c.at[0]], ssem).wait()

    plsc.subcore_barrier()
    pltpu.sync_copy(h2_sh.at[pl.ds(row0, RPT)],
                    h2p_hbm.at[c, pl.ds(row0, RPT)])


_sc_edge = pl.kernel(
    _sc_body,
    out_type=jax.ShapeDtypeStruct((2, NP, DE), jnp.float32),
    mesh=plsc.VectorSubcoreMesh(core_axis_name="c", subcore_axis_name="s"),
    scratch_types=[
        pltpu.VMEM((NP,), jnp.float32),
        pltpu.VMEM((3, BATCH), jnp.int32),
        pltpu.VMEM((3, BATCH), jnp.int32),
        pltpu.VMEM((2, BATCH, DE), jnp.float32),
        pltpu.VMEM_SHARED((NP, DE), jnp.float32),
        pltpu.SemaphoreType.DMA,
        pltpu.SemaphoreType.DMA,
        pltpu.SemaphoreType.DMA,
    ],
    compiler_params=pltpu.CompilerParams(needs_layout_passes=False,
                                         use_tc_tiling_on_sc=False),
)


def _impl(x, edge_index, batch, enc_W, enc_b, W, a_src, a_dst, b, ln_w, ln_b,
          dec_W, dec_b):
    pad = N + (jnp.arange(EPAD - E - N, dtype=jnp.int32) % (NP - N))
    loops = jnp.arange(N, dtype=jnp.int32)
    src = jnp.concatenate([edge_index[0].astype(jnp.int32), loops, pad])
    dst = jnp.concatenate([edge_index[1].astype(jnp.int32), loops, pad])
    src = src.reshape(NT, NBATCH, BATCH)
    dst = dst.reshape(NT, NBATCH, BATCH)

    r1 = lambda v: v.reshape(1, D)

    h0, g0, alad0 = _enc_call(x, enc_W, r1(enc_b), W[0], r1(a_src[0]),
                              r1(a_dst[0]))
    h2p0 = _sc_edge(g0, alad0, src, dst)
    h1, g1, alad1 = _mid_call(h2p0, h0, r1(b[0]), r1(ln_w[0]), r1(ln_b[0]),
                              W[1], r1(a_src[1]), r1(a_dst[1]))
    h2p1 = _sc_edge(g1, alad1, src, dst)
    out = _fin_call(h2p1, h1, r1(b[1]), r1(ln_w[1]), r1(ln_b[1]), dec_W,
                    r1(dec_b))
    return out.reshape(D)


kernel = jax.jit(_impl)
